# baseline (speedup 1.0000x reference)
import jax
import jax.numpy as jnp
from jax import lax
from jax.experimental import pallas as pl
from jax.experimental.pallas import tpu as pltpu

_NCHUNKS = 4


def _mlp_kernel(x_ref, w1_hbm, b1_ref, w2_hbm, b2_ref, o_ref,
                w1_ref, w2_ref, h_ref, sems):
    i = pl.program_id(0)
    H = w1_ref.shape[0]
    HC = H // _NCHUNKS

    def w1_chunk_copy(c):
        return pltpu.make_async_copy(
            w1_hbm.at[pl.ds(c * HC, HC), :], w1_ref.at[pl.ds(c * HC, HC), :],
            sems.at[c])

    w2_copy = pltpu.make_async_copy(w2_hbm.at[...], w2_ref.at[...],
                                    sems.at[_NCHUNKS])

    @pl.when(i == 0)
    def _start_weight_dmas():
        for c in range(_NCHUNKS):
            w1_chunk_copy(c).start()
        w2_copy.start()

    x = x_ref[...]
    for c in range(_NCHUNKS):
        @pl.when(i == 0)
        def _wait_w1_chunk(c=c):
            w1_chunk_copy(c).wait()
        h = lax.dot_general(
            x, w1_ref[pl.ds(c * HC, HC), :],
            dimension_numbers=(((1,), (1,)), ((), ())),
            preferred_element_type=jnp.float32,
        )
        h_ref[:, pl.ds(c * HC, HC)] = jnp.maximum(
            h + b1_ref[:, pl.ds(c * HC, HC)], 0.0).astype(jnp.bfloat16)

    @pl.when(i == 0)
    def _wait_w2():
        w2_copy.wait()

    y = lax.dot_general(
        h_ref[...], w2_ref[...],
        dimension_numbers=(((1,), (1,)), ((), ())),
        preferred_element_type=jnp.float32,
    )
    o_ref[...] = y + b2_ref[...]


def kernel(x, w1, b1, w2, b2):
    B, Din = x.shape
    H = w1.shape[0]
    O = w2.shape[0]

    TB = 512
    B_pad = ((B + TB - 1) // TB) * TB
    xp = jnp.pad(x, ((0, B_pad - B), (0, 0))) if B_pad != B else x
    b1_2d = b1.reshape(1, H)
    b2_2d = b2.reshape(1, O)

    out = pl.pallas_call(
        _mlp_kernel,
        out_shape=jax.ShapeDtypeStruct((B_pad, O), jnp.float32),
        grid=(B_pad // TB,),
        in_specs=[
            pl.BlockSpec((TB, Din), lambda i: (i, 0)),
            pl.BlockSpec(memory_space=pl.ANY),
            pl.BlockSpec((1, H), lambda i: (0, 0)),
            pl.BlockSpec(memory_space=pl.ANY),
            pl.BlockSpec((1, O), lambda i: (0, 0)),
        ],
        out_specs=pl.BlockSpec((TB, O), lambda i: (i, 0)),
        scratch_shapes=[
            pltpu.VMEM((H, Din), jnp.float32),
            pltpu.VMEM((O, H), jnp.float32),
            pltpu.VMEM((TB, H), jnp.bfloat16),
            pltpu.SemaphoreType.DMA((_NCHUNKS + 1,)),
        ],
        compiler_params=pltpu.CompilerParams(
            dimension_semantics=("arbitrary",),
        ),
    )(xp, w1, b1_2d, w2, b2_2d)
    return out[:B] if B_pad != B else out


# speedup vs baseline: 1.0987x; 1.0987x over previous
import jax
import jax.numpy as jnp
from jax import lax
from jax.experimental import pallas as pl
from jax.experimental.pallas import tpu as pltpu

_NCHUNKS = 4


def _mlp_kernel(x_ref, w1_hbm, b1_ref, w2_hbm, b2_ref, o_ref,
                w1_ref, w2_ref, h_ref, sems):
    i = pl.program_id(0)
    H = w1_ref.shape[0]
    HC = H // _NCHUNKS

    def w1_chunk_copy(c):
        return pltpu.make_async_copy(
            w1_hbm.at[pl.ds(c * HC, HC), :], w1_ref.at[pl.ds(c * HC, HC), :],
            sems.at[c])

    def w2_copy():
        return pltpu.make_async_copy(w2_hbm.at[...], w2_ref.at[...],
                                     sems.at[_NCHUNKS])

    def fc1_chunk(c):
        h = lax.dot_general(
            x_ref[...], w1_ref[pl.ds(c * HC, HC), :],
            dimension_numbers=(((1,), (1,)), ((), ())),
            preferred_element_type=jnp.float32,
        )
        h_ref[:, pl.ds(c * HC, HC)] = jnp.maximum(
            h + b1_ref[:, pl.ds(c * HC, HC)], 0.0).astype(jnp.bfloat16)

    @pl.when(i == 0)
    def _first_step_fc1():
        for c in range(_NCHUNKS):
            w1_chunk_copy(c).start()
        w2_copy().start()
        for c in range(_NCHUNKS):
            w1_chunk_copy(c).wait()
            fc1_chunk(c)
        w2_copy().wait()

    @pl.when(i != 0)
    def _steady_state_fc1():
        h = lax.dot_general(
            x_ref[...], w1_ref[...],
            dimension_numbers=(((1,), (1,)), ((), ())),
            preferred_element_type=jnp.float32,
        )
        h_ref[...] = jnp.maximum(h + b1_ref[...], 0.0).astype(jnp.bfloat16)

    y = lax.dot_general(
        h_ref[...], w2_ref[...],
        dimension_numbers=(((1,), (1,)), ((), ())),
        preferred_element_type=jnp.float32,
    )
    o_ref[...] = y + b2_ref[...]


def kernel(x, w1, b1, w2, b2):
    B, Din = x.shape
    H = w1.shape[0]
    O = w2.shape[0]

    TB = 512
    B_pad = ((B + TB - 1) // TB) * TB
    xp = jnp.pad(x, ((0, B_pad - B), (0, 0))) if B_pad != B else x
    b1_2d = b1.reshape(1, H)
    b2_2d = b2.reshape(1, O)

    out = pl.pallas_call(
        _mlp_kernel,
        out_shape=jax.ShapeDtypeStruct((B_pad, O), jnp.float32),
        grid=(B_pad // TB,),
        in_specs=[
            pl.BlockSpec((TB, Din), lambda i: (i, 0)),
            pl.BlockSpec(memory_space=pl.ANY),
            pl.BlockSpec((1, H), lambda i: (0, 0)),
            pl.BlockSpec(memory_space=pl.ANY),
            pl.BlockSpec((1, O), lambda i: (0, 0)),
        ],
        out_specs=pl.BlockSpec((TB, O), lambda i: (i, 0)),
        scratch_shapes=[
            pltpu.VMEM((H, Din), jnp.float32),
            pltpu.VMEM((O, H), jnp.float32),
            pltpu.VMEM((TB, H), jnp.bfloat16),
            pltpu.SemaphoreType.DMA((_NCHUNKS + 1,)),
        ],
        compiler_params=pltpu.CompilerParams(
            dimension_semantics=("arbitrary",),
        ),
    )(xp, w1, b1_2d, w2, b2_2d)
    return out[:B] if B_pad != B else out


# speedup vs baseline: 1.1168x; 1.0165x over previous
import jax
import jax.numpy as jnp
from jax import lax
from jax.experimental import pallas as pl
from jax.experimental.pallas import tpu as pltpu

_NCHUNKS = 4


def _dot_t(a, b):
    return lax.dot_general(
        a, b,
        dimension_numbers=(((1,), (1,)), ((), ())),
        preferred_element_type=jnp.float32,
    )


def _mlp_kernel(x_ref, w1_hbm, b1_ref, w2_hbm, b2_ref, o_ref,
                w1_ref, w2_ref, h_ref, sems):
    i = pl.program_id(0)
    H = w1_ref.shape[0]
    O = w2_ref.shape[0]
    HC = H // _NCHUNKS
    OC = O // _NCHUNKS

    def w1_chunk_copy(c):
        return pltpu.make_async_copy(
            w1_hbm.at[pl.ds(c * HC, HC), :], w1_ref.at[pl.ds(c * HC, HC), :],
            sems.at[c])

    def w2_chunk_copy(c):
        return pltpu.make_async_copy(
            w2_hbm.at[pl.ds(c * OC, OC), :], w2_ref.at[pl.ds(c * OC, OC), :],
            sems.at[_NCHUNKS + c])

    @pl.when(i == 0)
    def _first_step():
        for c in range(_NCHUNKS):
            w1_chunk_copy(c).start()
        for c in range(_NCHUNKS):
            w2_chunk_copy(c).start()
        for c in range(_NCHUNKS):
            w1_chunk_copy(c).wait()
            h = _dot_t(x_ref[...], w1_ref[pl.ds(c * HC, HC), :])
            h_ref[:, pl.ds(c * HC, HC)] = jnp.maximum(
                h + b1_ref[:, pl.ds(c * HC, HC)], 0.0)
        for c in range(_NCHUNKS):
            w2_chunk_copy(c).wait()
            y = _dot_t(h_ref[...], w2_ref[pl.ds(c * OC, OC), :])
            o_ref[:, pl.ds(c * OC, OC)] = y + b2_ref[:, pl.ds(c * OC, OC)]

    @pl.when(i != 0)
    def _steady_state():
        h = _dot_t(x_ref[...], w1_ref[...])
        h_ref[...] = jnp.maximum(h + b1_ref[...], 0.0)
        y = _dot_t(h_ref[...], w2_ref[...])
        o_ref[...] = y + b2_ref[...]


def kernel(x, w1, b1, w2, b2):
    B, Din = x.shape
    H = w1.shape[0]
    O = w2.shape[0]

    TB = 512
    B_pad = ((B + TB - 1) // TB) * TB
    xp = jnp.pad(x, ((0, B_pad - B), (0, 0))) if B_pad != B else x
    b1_2d = b1.reshape(1, H)
    b2_2d = b2.reshape(1, O)

    out = pl.pallas_call(
        _mlp_kernel,
        out_shape=jax.ShapeDtypeStruct((B_pad, O), jnp.float32),
        grid=(B_pad // TB,),
        in_specs=[
            pl.BlockSpec((TB, Din), lambda i: (i, 0)),
            pl.BlockSpec(memory_space=pl.ANY),
            pl.BlockSpec((1, H), lambda i: (0, 0)),
            pl.BlockSpec(memory_space=pl.ANY),
            pl.BlockSpec((1, O), lambda i: (0, 0)),
        ],
        out_specs=pl.BlockSpec((TB, O), lambda i: (i, 0)),
        scratch_shapes=[
            pltpu.VMEM((H, Din), jnp.float32),
            pltpu.VMEM((O, H), jnp.float32),
            pltpu.VMEM((TB, H), jnp.float32),
            pltpu.SemaphoreType.DMA((2 * _NCHUNKS,)),
        ],
        compiler_params=pltpu.CompilerParams(
            dimension_semantics=("arbitrary",),
        ),
    )(xp, w1, b1_2d, w2, b2_2d)
    return out[:B] if B_pad != B else out


# speedup vs baseline: 1.1429x; 1.0233x over previous
import jax
import jax.numpy as jnp
from jax import lax
from jax.experimental import pallas as pl
from jax.experimental.pallas import tpu as pltpu

_NCHUNKS = 4
_NCHUNKS2 = 2


def _dot_t(a, b):
    return lax.dot_general(
        a, b,
        dimension_numbers=(((1,), (1,)), ((), ())),
        preferred_element_type=jnp.float32,
    )


def _mlp_kernel(x_ref, w1_hbm, b1_ref, w2_hbm, b2_ref, o_ref,
                w1_ref, w2_ref, h_ref, sems):
    i = pl.program_id(0)
    H = w1_ref.shape[0]
    O = w2_ref.shape[0]
    HC = H // _NCHUNKS
    OC = O // _NCHUNKS2

    def w1_chunk_copy(c):
        return pltpu.make_async_copy(
            w1_hbm.at[pl.ds(c * HC, HC), :], w1_ref.at[pl.ds(c * HC, HC), :],
            sems.at[c])

    def w2_chunk_copy(c):
        return pltpu.make_async_copy(
            w2_hbm.at[pl.ds(c * OC, OC), :], w2_ref.at[pl.ds(c * OC, OC), :],
            sems.at[_NCHUNKS + c])

    @pl.when(i == 0)
    def _first_step():
        for c in range(_NCHUNKS):
            w1_chunk_copy(c).start()
        for c in range(_NCHUNKS2):
            w2_chunk_copy(c).start()
        for c in range(_NCHUNKS):
            w1_chunk_copy(c).wait()
            h = _dot_t(x_ref[...], w1_ref[pl.ds(c * HC, HC), :])
            h_ref[:, pl.ds(c * HC, HC)] = jnp.maximum(
                h + b1_ref[:, pl.ds(c * HC, HC)], 0.0)
        for c in range(_NCHUNKS2):
            w2_chunk_copy(c).wait()
            y = _dot_t(h_ref[...], w2_ref[pl.ds(c * OC, OC), :])
            o_ref[:, pl.ds(c * OC, OC)] = y + b2_ref[:, pl.ds(c * OC, OC)]

    @pl.when(i != 0)
    def _steady_state():
        h = _dot_t(x_ref[...], w1_ref[...])
        h_ref[...] = jnp.maximum(h + b1_ref[...], 0.0)
        y = _dot_t(h_ref[...], w2_ref[...])
        o_ref[...] = y + b2_ref[...]


def kernel(x, w1, b1, w2, b2):
    B, Din = x.shape
    H = w1.shape[0]
    O = w2.shape[0]

    TB = 512
    B_pad = ((B + TB - 1) // TB) * TB
    xp = jnp.pad(x, ((0, B_pad - B), (0, 0))) if B_pad != B else x
    b1_2d = b1.reshape(1, H)
    b2_2d = b2.reshape(1, O)

    out = pl.pallas_call(
        _mlp_kernel,
        out_shape=jax.ShapeDtypeStruct((B_pad, O), jnp.float32),
        grid=(B_pad // TB,),
        in_specs=[
            pl.BlockSpec((TB, Din), lambda i: (i, 0)),
            pl.BlockSpec(memory_space=pl.ANY),
            pl.BlockSpec((1, H), lambda i: (0, 0)),
            pl.BlockSpec(memory_space=pl.ANY),
            pl.BlockSpec((1, O), lambda i: (0, 0)),
        ],
        out_specs=pl.BlockSpec((TB, O), lambda i: (i, 0)),
        scratch_shapes=[
            pltpu.VMEM((H, Din), jnp.float32),
            pltpu.VMEM((O, H), jnp.float32),
            pltpu.VMEM((TB, H), jnp.float32),
            pltpu.SemaphoreType.DMA((_NCHUNKS + _NCHUNKS2,)),
        ],
        compiler_params=pltpu.CompilerParams(
            dimension_semantics=("arbitrary",),
        ),
    )(xp, w1, b1_2d, w2, b2_2d)
    return out[:B] if B_pad != B else out
